# Initial kernel scaffold; baseline (speedup 1.0000x reference)
import jax
import jax.numpy as jnp
from jax.experimental import pallas as pl


def kernel(input_ids, wte, wpe):
    raise NotImplementedError("write your pallas kernel here")


# speedup vs baseline: 1.3398x; 1.3398x over previous
import functools

import jax
import jax.numpy as jnp
from jax import lax
from jax.experimental import pallas as pl
from jax.experimental.pallas import tpu as pltpu
from jax.experimental.pallas import tpu_sc as plsc

VOCAB = 50257
N_POS = 2048
N_EMBD = 768
SEQ = 2048

_NC = 2
_NS = 16
_NW = _NC * _NS
_BPW = SEQ // _NW
_LANES = 16
_CHUNKS = N_EMBD // _LANES

_mesh = plsc.VectorSubcoreMesh(core_axis_name="c", subcore_axis_name="s")


@functools.partial(
    pl.kernel,
    out_type=jax.ShapeDtypeStruct((SEQ, N_EMBD), jnp.float32),
    mesh=_mesh,
    scratch_types=[
        pltpu.VMEM((_BPW,), jnp.int32),
        pltpu.VMEM((_BPW, N_EMBD), jnp.float32),
        pltpu.VMEM((_BPW, N_EMBD), jnp.float32),
        pltpu.SemaphoreType.DMA,
    ],
)
def _emb_lookup(wte_hbm, ids_hbm, wpe_hbm, out_hbm, ids_v, rows_v, wpe_v, sem):
    wid = lax.axis_index("s") * _NC + lax.axis_index("c")
    base = wid * _BPW

    pltpu.sync_copy(ids_hbm.at[pl.ds(base, _BPW)], ids_v)
    gather = pltpu.async_copy(wte_hbm.at[ids_v], rows_v, sem)
    pltpu.sync_copy(wpe_hbm.at[pl.ds(base, _BPW)], wpe_v)
    gather.wait()

    def add_row(r):
        for c in range(_CHUNKS):
            sl = pl.ds(c * _LANES, _LANES)
            rows_v[r, sl] += wpe_v[r, sl]

    lax.fori_loop(0, _BPW, lambda r, _: (add_row(r), 0)[1], 0)

    pltpu.sync_copy(rows_v, out_hbm.at[pl.ds(base, _BPW)])


def kernel(input_ids, wte, wpe):
    ids = input_ids.astype(jnp.int32)
    out = _emb_lookup(wte, ids, wpe)
    return out[None, :, :]
